# baseline (speedup 1.0000x reference)
import jax
import jax.numpy as jnp
from jax.experimental import pallas as pl


def _moe_dense_body(idx_ref, score_ref, x_ref, w_ref, b_ref, o_ref):
    e = pl.program_id(0)
    match = (idx_ref[...] == e).astype(jnp.float32)
    coeff = jnp.sum(score_ref[...] * match, axis=1, keepdims=True)
    y = jax.lax.dot_general(
        x_ref[...].astype(jnp.bfloat16), w_ref[0].astype(jnp.bfloat16),
        dimension_numbers=(((1,), (1,)), ((), ())),
        preferred_element_type=jnp.float32,
    )
    contrib = coeff * (y + b_ref[0])

    @pl.when(e == 0)
    def _():
        o_ref[...] = contrib

    @pl.when(e != 0)
    def _():
        o_ref[...] += contrib


def kernel(inp, gate_idx, gate_score, W, b):
    T, D = inp.shape
    E = W.shape[0]
    K = gate_idx.shape[1]

    return pl.pallas_call(
        _moe_dense_body,
        grid=(E,),
        in_specs=[
            pl.BlockSpec((T, K), lambda e: (0, 0)),
            pl.BlockSpec((T, K), lambda e: (0, 0)),
            pl.BlockSpec((T, D), lambda e: (0, 0)),
            pl.BlockSpec((1, D, D), lambda e: (e, 0, 0)),
            pl.BlockSpec((1, 1, D), lambda e: (e, 0, 0)),
        ],
        out_specs=pl.BlockSpec((T, D), lambda e: (0, 0)),
        out_shape=jax.ShapeDtypeStruct((T, D), jnp.float32),
    )(gate_idx, gate_score, inp, W, b.reshape(E, 1, D))


# speedup vs baseline: 1.4672x; 1.4672x over previous
import functools

import jax
import jax.numpy as jnp
from jax import lax
from jax.experimental import pallas as pl
from jax.experimental.pallas import tpu as pltpu
from jax.experimental.pallas import tpu_sc as plsc

NUM_E = 64
D = 1024
T = 2048
K = 2
R = T * K
TILE_R = 128
PR = R + NUM_E * 8
PAD_R = PR + 64
N_SUB = 16
RPT = R // N_SUB
GPT = PR // 32


def _it16():
    return lax.iota(jnp.int32, 16)


def _sc_route_dispatch(gate_flat, inp):
    mesh = plsc.VectorSubcoreMesh(core_axis_name="c", subcore_axis_name="s")

    @functools.partial(
        pl.kernel,
        out_type=[
            jax.ShapeDtypeStruct((PAD_R, D), jnp.float32),
            jax.ShapeDtypeStruct((128,), jnp.int32),
            jax.ShapeDtypeStruct((N_SUB, RPT), jnp.int32),
        ],
        mesh=mesh,
        scratch_types=[
            pltpu.VMEM((RPT,), jnp.int32),
            pltpu.VMEM((RPT,), jnp.int32),
            pltpu.VMEM((NUM_E,), jnp.int32),
            pltpu.VMEM((N_SUB, NUM_E), jnp.int32),
            pltpu.VMEM((NUM_E,), jnp.int32),
            pltpu.VMEM((128,), jnp.int32),
            pltpu.VMEM((RPT,), jnp.int32),
            pltpu.VMEM((N_SUB, RPT), jnp.int32),
            pltpu.VMEM((PR,), jnp.int32),
            pltpu.VMEM((48, D), jnp.float32),
            pltpu.VMEM_SHARED((N_SUB, NUM_E), jnp.int32),
            pltpu.VMEM_SHARED((N_SUB, RPT), jnp.int32),
            pltpu.SemaphoreType.DMA,
        ],
        compiler_params=pltpu.CompilerParams(needs_layout_passes=False),
    )
    def k(gate_hbm, inp_hbm, xs_hbm, starts_hbm, slots_hbm,
          ids, rank, counts, counts_all, offadj, starts_st, slots_l,
          slots_all, tok, rowbuf, counts_sh, slots_sh, sem):
        cid = lax.axis_index("c")
        lt = lax.axis_index("s")
        it = _it16()

        pltpu.sync_copy(gate_hbm.at[lt], ids)

        def expert_body(e, carry):
            prior = jnp.zeros((16,), jnp.int32)
            for j in range(16):
                idv = ids[pl.ds(16 * j, 16)]
                m = idv == e
                ranks = plsc.cumsum(m.astype(jnp.int32))
                plsc.store_scatter(rank, [it + 16 * j], prior + ranks - 1,
                                   mask=m)
                prior = prior + plsc.all_reduce_population_count(m)
            plsc.store_scatter(counts, [jnp.full((16,), e, jnp.int32)], prior,
                               mask=it == 0)
            return carry

        lax.fori_loop(0, NUM_E, expert_body, 0)

        pltpu.sync_copy(counts, counts_sh.at[lt])
        plsc.subcore_barrier()

        pltpu.sync_copy(counts_sh, counts_all)
        run = jnp.int32(0)
        for q in range(4):
            tot = jnp.zeros((16,), jnp.int32)
            pri = jnp.zeros((16,), jnp.int32)
            for r in range(N_SUB):
                row = counts_all[r, pl.ds(16 * q, 16)]
                tot = tot + row
                pri = pri + jnp.where(r < lt, row, 0)
            ptot = ((tot + 7) >> 3) << 3
            excl = plsc.cumsum(ptot) - ptot
            starts_q = excl + run
            offadj[pl.ds(16 * q, 16)] = starts_q + pri
            starts_st[pl.ds(16 * q, 16)] = starts_q
            run = run + jnp.sum(ptot)
        starts_st[pl.ds(64, 16)] = jnp.where(it == 0, run, 0)
        for q in range(5, 8):
            starts_st[pl.ds(16 * q, 16)] = jnp.zeros((16,), jnp.int32)

        @pl.when(jnp.logical_and(cid == 0, lt == 0))
        def _():
            pltpu.sync_copy(starts_st, starts_hbm)

        for j in range(16):
            idv = ids[pl.ds(16 * j, 16)]
            rv = rank[pl.ds(16 * j, 16)]
            slots_l[pl.ds(16 * j, 16)] = plsc.load_gather(offadj, [idv]) + rv
        pltpu.sync_copy(slots_l, slots_sh.at[lt])

        @pl.when(cid == 0)
        def _():
            pltpu.sync_copy(slots_l, slots_hbm.at[lt])

        plsc.subcore_barrier()

        for z in range(PR // 16):
            tok[pl.ds(16 * z, 16)] = jnp.zeros((16,), jnp.int32)
        pltpu.sync_copy(slots_sh, slots_all)
        for r in range(N_SUB):
            for jj in range(RPT // 16):
                sv = slots_all[r, pl.ds(16 * jj, 16)]
                tokv = (it + (RPT * r + 16 * jj)) >> 1
                plsc.store_scatter(tok, [sv], tokv)

        base_row = GPT * (cid * N_SUB + lt)
        for c in range(3):
            off = base_row + 48 * c
            pltpu.async_copy(inp_hbm.at[tok.at[pl.ds(off, 48)]], rowbuf,
                             sem).wait()
            pltpu.sync_copy(rowbuf, xs_hbm.at[pl.ds(off, 48)])

    return k(gate_flat, inp)


def _tc_gemm_body(starts_ref, x_ref, w_ref, b_ref, y_ref):
    e = pl.program_id(0)
    s0 = starts_ref[e]
    n = (starts_ref[e + 1] - s0 + TILE_R - 1) // TILE_R
    wmat = w_ref[0].astype(jnp.bfloat16)
    bias = b_ref[0]

    def tile_body(i, carry):
        base = pl.multiple_of(s0 + i * TILE_R, 8)
        xt = x_ref[pl.ds(base, TILE_R), :].astype(jnp.bfloat16)
        acc = jax.lax.dot_general(
            xt, wmat, dimension_numbers=(((1,), (1,)), ((), ())),
            preferred_element_type=jnp.float32)
        y_ref[pl.ds(base, TILE_R), :] = acc + bias
        return carry

    lax.fori_loop(0, n, tile_body, 0)


def _tc_gemm(starts, x_sorted, W, b):
    grid_spec = pltpu.PrefetchScalarGridSpec(
        num_scalar_prefetch=1,
        grid=(NUM_E,),
        in_specs=[
            pl.BlockSpec((PAD_R, D), lambda e, s: (0, 0)),
            pl.BlockSpec((1, D, D), lambda e, s: (e, 0, 0)),
            pl.BlockSpec((1, 1, D), lambda e, s: (e, 0, 0)),
        ],
        out_specs=pl.BlockSpec((PAD_R, D), lambda e, s: (0, 0)),
    )
    return pl.pallas_call(
        _tc_gemm_body,
        grid_spec=grid_spec,
        out_shape=jax.ShapeDtypeStruct((PAD_R, D), jnp.float32),
    )(starts, x_sorted, W, b.reshape(NUM_E, 1, D))


def _sc_combine(slots_flat, score_flat, y):
    mesh = plsc.VectorSubcoreMesh(core_axis_name="c", subcore_axis_name="s")

    @functools.partial(
        pl.kernel,
        out_type=jax.ShapeDtypeStruct((T, D), jnp.float32),
        mesh=mesh,
        scratch_types=[
            pltpu.VMEM((128,), jnp.int32),
            pltpu.VMEM((128,), jnp.float32),
            pltpu.VMEM((64, D), jnp.float32),
            pltpu.VMEM((32, D), jnp.float32),
            pltpu.SemaphoreType.DMA,
        ],
        compiler_params=pltpu.CompilerParams(needs_layout_passes=False),
    )
    def k(slots_hbm, score_hbm, y_hbm, out_hbm, sl, sc, ybuf, obuf, sem):
        wid = lax.axis_index("c") * N_SUB + lax.axis_index("s")
        r = wid // 2
        c0 = 128 * (wid % 2)
        pltpu.sync_copy(slots_hbm.at[r, pl.ds(c0, 128)], sl)
        pltpu.sync_copy(score_hbm.at[r, pl.ds(c0, 128)], sc)
        for half in range(2):
            pltpu.async_copy(y_hbm.at[sl.at[pl.ds(64 * half, 64)]], ybuf,
                             sem).wait()

            def tok_body(jj, carry):
                f = 64 * half + 2 * jj
                s0 = plsc.load_gather(sc, [jnp.full((16,), f, jnp.int32)])
                s1 = plsc.load_gather(sc, [jnp.full((16,), f + 1, jnp.int32)])
                for d in range(D // 16):
                    y0 = ybuf[2 * jj, pl.ds(16 * d, 16)]
                    y1 = ybuf[2 * jj + 1, pl.ds(16 * d, 16)]
                    obuf[jj, pl.ds(16 * d, 16)] = s0 * y0 + s1 * y1
                return carry

            lax.fori_loop(0, 32, tok_body, 0)
            pltpu.sync_copy(obuf, out_hbm.at[pl.ds(64 * wid + 32 * half, 32)])

    return k(slots_flat, score_flat, y)


def kernel(inp, gate_idx, gate_score, W, b):
    gate_flat = gate_idx.reshape(N_SUB, RPT)
    score_flat = gate_score.reshape(N_SUB, RPT)
    x_sorted, starts, slots = _sc_route_dispatch(gate_flat, inp)
    y = _tc_gemm(starts, x_sorted, W, b)
    return _sc_combine(slots, score_flat, y)
